# baseline (speedup 1.0000x reference)
import functools

import jax
import jax.numpy as jnp
from jax import lax
from jax.experimental import pallas as pl
from jax.experimental.pallas import tpu as pltpu
from jax.experimental.pallas import tpu_sc as plsc

VOCAB = 1000000
EMBED = 64
BATCH = 16384

_info = plsc.get_sparse_core_info()
_NC = _info.num_cores
_NS = _info.num_subcores
_NW = _NC * _NS
_RPW = BATCH // _NW
_CH = 128
_NCHUNK = _RPW // _CH

_mesh = plsc.VectorSubcoreMesh(core_axis_name="c", subcore_axis_name="s")


@functools.partial(
    pl.kernel,
    mesh=_mesh,
    out_type=(
        jax.ShapeDtypeStruct((EMBED * BATCH,), jnp.float32),
        jax.ShapeDtypeStruct((EMBED * BATCH,), jnp.float32),
    ),
    scratch_types=[
        pltpu.VMEM((_CH,), jnp.int32),
        pltpu.VMEM((_CH,), jnp.int32),
        pltpu.VMEM((_CH, 128), jnp.float32),
        pltpu.VMEM((EMBED * _CH,), jnp.float32),
        pltpu.SemaphoreType.DMA,
    ],
    compiler_params=pltpu.CompilerParams(needs_layout_passes=False),
)
def _lookup(center_hbm, context_hbm, table2_hbm, out_c_hbm, out_x_hbm,
            idx_rows, idx_m, dst, stg, sem):
    wid = lax.axis_index("s") * _NC + lax.axis_index("c")
    iota16 = lax.iota(jnp.int32, 16)

    for in_ref, out_ref in ((center_hbm, out_c_hbm), (context_hbm, out_x_hbm)):

        @pl.loop(0, _NCHUNK)
        def chunk_loop(ch):
            s_prime = wid * _NCHUNK + ch
            base_row = s_prime * _CH
            pltpu.sync_copy(in_ref.at[pl.ds(base_row, _CH)], idx_rows)
            for rb in range(_CH // 16):
                v = idx_rows[pl.ds(rb * 16, 16)]
                idx_m[pl.ds(rb * 16, 16)] = v >> 1
            pltpu.async_copy(table2_hbm.at[idx_m], dst, sem).wait()
            for rb in range(_CH // 16):
                v = idx_rows[pl.ds(rb * 16, 16)]
                rows = iota16 + (rb * 16)
                half = (v & 1) << 6
                for c in range(EMBED):
                    stg[pl.ds(c * _CH + rb * 16, 16)] = plsc.load_gather(
                        dst, [rows, half + c]
                    )
            for k in range(EMBED // 8):
                pltpu.sync_copy(
                    stg.at[pl.ds(k * 1024, 1024)],
                    out_ref.at[pl.ds(k * 131072 + s_prime * 1024, 1024)],
                )


def kernel(center, context, word_embeds):
    buf_c, buf_x = _lookup(center, context, word_embeds.reshape(VOCAB // 2, 128))

    def fix(buf):
        return buf.reshape(8, 128, 8, 128).transpose(1, 3, 0, 2).reshape(BATCH, EMBED)

    return (fix(buf_c), fix(buf_x))


# speedup vs baseline: 1.0206x; 1.0206x over previous
import functools

import jax
import jax.numpy as jnp
from jax import lax
from jax.experimental import pallas as pl
from jax.experimental.pallas import tpu as pltpu
from jax.experimental.pallas import tpu_sc as plsc

VOCAB = 1000000
EMBED = 64
BATCH = 16384

_info = plsc.get_sparse_core_info()
_NC = _info.num_cores
_NS = _info.num_subcores
_NW = _NC * _NS
_BPW = BATCH // _NW

_mesh = plsc.VectorSubcoreMesh(core_axis_name="c", subcore_axis_name="s")


@functools.partial(
    pl.kernel,
    mesh=_mesh,
    out_type=(
        jax.ShapeDtypeStruct((EMBED * BATCH,), jnp.float32),
        jax.ShapeDtypeStruct((EMBED * BATCH,), jnp.float32),
    ),
    scratch_types=[
        pltpu.VMEM((_BPW,), jnp.int32),
        pltpu.VMEM((_BPW,), jnp.int32),
        pltpu.VMEM((_BPW, EMBED), jnp.float32),
        pltpu.VMEM((_BPW, EMBED), jnp.float32),
        pltpu.VMEM((_BPW * EMBED,), jnp.float32),
        pltpu.SemaphoreType.DMA,
        pltpu.SemaphoreType.DMA,
    ],
    compiler_params=pltpu.CompilerParams(
        use_tc_tiling_on_sc=False, needs_layout_passes=False
    ),
)
def _lookup(center_hbm, context_hbm, table_hbm, out_c_hbm, out_x_hbm,
            idx_c, idx_x, rows_c, rows_x, stg, sem_c, sem_x):
    wid = lax.axis_index("s") * _NC + lax.axis_index("c")
    base = wid * _BPW
    iota16 = lax.iota(jnp.int32, 16)
    pltpu.sync_copy(center_hbm.at[pl.ds(base, _BPW)], idx_c)
    pltpu.sync_copy(context_hbm.at[pl.ds(base, _BPW)], idx_x)
    cp_c = pltpu.async_copy(table_hbm.at[idx_c], rows_c, sem_c)
    cp_x = pltpu.async_copy(table_hbm.at[idx_x], rows_x, sem_x)

    for rows, out_ref, cp in ((rows_c, out_c_hbm, cp_c), (rows_x, out_x_hbm, cp_x)):
        cp.wait()

        @pl.loop(0, _BPW // 16)
        def rb_loop(rb):
            rows16 = iota16 + rb * 16
            dyn = (rb >> 3) * 1024 + (rb & 7) * 16
            for c in range(EMBED):
                k, j = c // 8, c % 8
                stg[pl.ds(dyn + k * 4096 + j * 128, 16)] = plsc.load_gather(
                    rows, [rows16, iota16 * 0 + c]
                )

        for k in range(EMBED // 8):
            for ch in range(_BPW // 128):
                pltpu.sync_copy(
                    stg.at[pl.ds((k * 4 + ch) * 1024, 1024)],
                    out_ref.at[pl.ds((k * 128 + wid * 4 + ch) * 1024, 1024)],
                )


def kernel(center, context, word_embeds):
    buf_c, buf_x = _lookup(center, context, word_embeds)

    def fix(buf):
        return buf.reshape(8, 128, 8, 128).transpose(1, 3, 0, 2).reshape(BATCH, EMBED)

    return (fix(buf_c), fix(buf_x))
